# Initial kernel scaffold; baseline (speedup 1.0000x reference)
import jax
import jax.numpy as jnp
from jax.experimental import pallas as pl


def kernel(x, edge_index, edge_attr, batch, c1_init_w, c1_w, c1_root_w, c1_b, c2_init_w, c2_w, c2_root_w, c2_b, l1_w, l1_b, bn1_g, bn1_b, l2_w, l2_b, bn2_g, bn2_b, l3_w, l3_b, bn3_g, bn3_b, lo_w, lo_b):
    raise NotImplementedError("write your pallas kernel here")


# speedup vs baseline: 1.1814x; 1.1814x over previous
import functools

import jax
import jax.numpy as jnp
from jax import lax
from jax.experimental import pallas as pl
from jax.experimental.pallas import tpu as pltpu
from jax.experimental.pallas import tpu_sc as plsc

N = 10000
E = 160000
G = 64
FIN = 4
H = 256
C = 10
KS = 3
TL = 3
HC = H // 2

NC, NS = 2, 16
NW = NC * NS

EP = 163840
CHUNKS = EP // 128
NSUPT = CHUNKS // 8
SUPC_DEG = NSUPT // NW
SUPC = NSUPT // NS
SUP = 4
NP = 10240
ROWS_T = NP // NS

BM = 1000
MB = N // BM

_GDN = lax.GatherDimensionNumbers(offset_dims=(), collapsed_slice_dims=(0,),
                                  start_index_map=(0,))


def _deg_body(col3d, ew3d, out, colbuf, ewbuf, zbuf, acc):
    cid = lax.axis_index("c")
    sid = lax.axis_index("s")
    wid = cid * NS + sid

    @pl.when(sid == 0)
    def _():
        def zb(i, _):
            zbuf[pl.ds(i * 16, 16)] = jnp.zeros((16,), jnp.float32)
            return 0
        lax.fori_loop(0, N // 16, zb, 0)
        pltpu.sync_copy(zbuf, acc)
    plsc.subcore_barrier()

    def body(si, _):
        sc = wid * SUPC_DEG + si
        for j in range(8):
            pltpu.sync_copy(col3d.at[sc].at[j], colbuf)
            pltpu.sync_copy(ew3d.at[sc].at[j], ewbuf)
            pltpu.sync_copy(ewbuf, acc.at[colbuf], add=True)
        return 0
    lax.fori_loop(0, SUPC_DEG, body, 0)
    plsc.subcore_barrier()

    @pl.when(sid == 0)
    def _():
        pltpu.sync_copy(acc, out.at[cid].at[0])


@functools.lru_cache(maxsize=None)
def _deg_kernel():
    mesh = plsc.VectorSubcoreMesh(core_axis_name="c", subcore_axis_name="s")
    return pl.kernel(
        _deg_body, mesh=mesh,
        out_type=jax.ShapeDtypeStruct((NC, 1, N), jnp.float32),
        scratch_types=[
            pltpu.VMEM((128,), jnp.int32),
            pltpu.VMEM((128,), jnp.float32),
            pltpu.VMEM((N,), jnp.float32),
            pltpu.VMEM_SHARED((N,), jnp.float32),
        ],
    )


def _deg_call(col2d, ew2d):
    return _deg_kernel()(col2d, ew2d)


NCH = CHUNKS // NS


PH = 5120
ACCR = 6144
AZT = ACCR // NS
PWT = PH // NS


def _prop_body(h6, row3d, colp, ew3d, out, rowall, colall, ewall, rows3,
               zbuf, acc, gsem):
    cid = lax.axis_index("c")
    sid = lax.axis_index("s")

    def zb(i, _):
        r = i >> 3
        f = (i & 7) * 16
        zbuf[r, pl.ds(f, 16)] = jnp.zeros((16,), jnp.float32)
        return 0
    lax.fori_loop(0, 128 * 8, zb, 0)

    pltpu.sync_copy(row3d.at[pl.ds(sid * SUPC, SUPC)], rowall)
    pltpu.sync_copy(ew3d.at[pl.ds(sid * SUPC, SUPC)], ewall)

    def gather_of(k, ch, par):
        return pltpu.make_async_copy(
            h6.at[k * NC + cid].at[rowall.at[ch >> 3].at[ch & 7]],
            rows3.at[par], gsem.at[par])

    def scatter_sync(ch, par):
        pltpu.sync_copy(rows3.at[par], acc.at[colall.at[ch >> 3].at[ch & 7]],
                        add=True)

    def zero_acc():
        for z in range(AZT // 128):
            pltpu.sync_copy(zbuf, acc.at[pl.ds(sid * AZT + z * 128, 128)])

    def stage_cols(p):
        pltpu.sync_copy(colp.at[p].at[pl.ds(sid * SUPC, SUPC)], colall)

    def writeback(k, p):
        pltpu.sync_copy(
            acc.at[pl.ds(sid * PWT, PWT)],
            out.at[k * NC + cid].at[pl.ds(p * PH + sid * PWT, PWT)])

    zero_acc()
    stage_cols(0)
    plsc.subcore_barrier()
    gather_of(0, 0, 0).start()

    def k_body(k, _):
        def p_body(p, _):
            first = jnp.logical_and(k == 0, p == 0)

            @pl.when(jnp.logical_not(first))
            def _():
                plsc.subcore_barrier()
                pk = jnp.where(p > 0, k, k - 1)
                pp = jnp.where(p > 0, p - 1, 1)
                writeback(pk, pp)
                plsc.subcore_barrier()
                zero_acc()
                stage_cols(p)
                plsc.subcore_barrier()

            def ch_body(ch, _):
                par = lax.rem(ch, 2)
                nxt = 1 - par
                wrap = ch == NCH - 1
                last = jnp.logical_and(
                    jnp.logical_and(k == KS - 1, p == NC - 1), wrap)
                nk = jnp.where(jnp.logical_and(wrap, p == NC - 1), k + 1, k)
                nch = jnp.where(wrap, 0, ch + 1)

                @pl.when(jnp.logical_not(last))
                def _():
                    gather_of(nk, nch, nxt).start()

                gather_of(k, ch, par).wait()

                def grp_body(gi, _):
                    ewg = ewall[ch >> 3, ch & 7, pl.ds(gi * 16, 16)]
                    for lane in range(16):
                        ewv = lax.gather(
                            ewg, jnp.full((16, 1), lane, jnp.int32), _GDN,
                            (1,),
                            mode=lax.GatherScatterMode.PROMISE_IN_BOUNDS)
                        e = gi * 16 + lane
                        for f in range(8):
                            rows3[par, e, pl.ds(f * 16, 16)] = (
                                rows3[par, e, pl.ds(f * 16, 16)] * ewv)
                    return 0
                lax.fori_loop(0, 8, grp_body, 0)

                scatter_sync(ch, par)
                return 0
            lax.fori_loop(0, NCH, ch_body, 0)
            return 0
        lax.fori_loop(0, NC, p_body, 0)
        return 0
    lax.fori_loop(0, KS, k_body, 0)

    plsc.subcore_barrier()
    writeback(KS - 1, NC - 1)
    plsc.subcore_barrier()


@functools.lru_cache(maxsize=None)
def _prop_kernel():
    mesh = plsc.VectorSubcoreMesh(core_axis_name="c", subcore_axis_name="s")
    return pl.kernel(
        _prop_body, mesh=mesh,
        out_type=jax.ShapeDtypeStruct((KS * NC, NP, HC), jnp.float32),
        scratch_types=[
            pltpu.VMEM((SUPC, 8, 128), jnp.int32),
            pltpu.VMEM((SUPC, 8, 128), jnp.int32),
            pltpu.VMEM((SUPC, 8, 128), jnp.float32),
            pltpu.VMEM((2, 128, HC), jnp.float32),
            pltpu.VMEM((128, HC), jnp.float32),
            pltpu.VMEM_SHARED((ACCR, HC), jnp.float32),
            pltpu.SemaphoreType.DMA((2,)),
        ],
    )


def _prop_call(h6, row3d, colp, ew3d):
    agg = _prop_kernel()(h6.reshape(KS * NC, N, HC), row3d, colp, ew3d)
    return agg.reshape(KS, NC, NP, HC)


def _dis_kernel(deg_ref, o_ref):
    d = jnp.sum(deg_ref[...], axis=0, keepdims=True)
    o_ref[...] = jnp.where(d > 0, lax.rsqrt(jnp.maximum(d, 1e-30)), 0.0)


def _dis(deg2):
    o = pl.pallas_call(
        _dis_kernel,
        out_shape=jax.ShapeDtypeStruct((1, N), jnp.float32),
    )(deg2)
    return o.reshape(N, 1)


def _front_kernel(a_ref, w_ref, b_ref, d_ref, o_ref, acc, *, kc_n, nsc):
    kc = pl.program_id(2)

    @pl.when(kc == 0)
    def _():
        acc[...] = jnp.zeros_like(acc)

    acc[...] += jnp.dot(a_ref[0], w_ref[0],
                        preferred_element_type=jnp.float32)

    @pl.when(kc == kc_n - 1)
    def _():
        n = pl.program_id(1)
        r = acc[...] + b_ref[0]
        r = jnp.where(n < nsc, r * d_ref[...], r)
        o_ref[0] = r


def _front(a3, w3, bias24, dis):
    kc_n, kb = a3.shape[0], a3.shape[2]
    return pl.pallas_call(
        functools.partial(_front_kernel, kc_n=kc_n, nsc=2 * KS),
        grid=(MB, 24, kc_n),
        in_specs=[
            pl.BlockSpec((1, BM, kb), lambda m, n, kc: (kc, m, 0)),
            pl.BlockSpec((1, kb, 128), lambda m, n, kc: (kc, 0, n)),
            pl.BlockSpec((1, 1, 128), lambda m, n, kc: (n, 0, 0)),
            pl.BlockSpec((BM, 1), lambda m, n, kc: (m, 0)),
        ],
        out_specs=pl.BlockSpec((1, BM, 128), lambda m, n, kc: (n, m, 0)),
        out_shape=jax.ShapeDtypeStruct((24, N, 128), jnp.float32),
        scratch_shapes=[pltpu.VMEM((BM, 128), jnp.float32)],
    )(a3, w3, bias24, dis)


def _layer_kernel(a_ref, r_ref, w_ref, d_ref, o_ref, acc):
    kc = pl.program_id(3)
    d = d_ref[...]
    bin_ = jnp.maximum(d * a_ref[0, 0] + r_ref[0, 0], 0.0)
    prod = jnp.dot(bin_, w_ref[0], preferred_element_type=jnp.float32)

    @pl.when(kc == 0)
    def _():
        acc[...] = prod

    @pl.when(kc == 1)
    def _():
        o_ref[0, 0] = d * (acc[...] + prod)


def _layer_mm(agg, root, w, dis):
    return pl.pallas_call(
        _layer_kernel,
        grid=(KS, MB, 2, 2),
        in_specs=[
            pl.BlockSpec((1, 1, BM, 128), lambda k, m, n, kc: (k, kc, m, 0)),
            pl.BlockSpec((1, 1, BM, 128), lambda k, m, n, kc: (k, kc, m, 0)),
            pl.BlockSpec((1, 128, 128), lambda k, m, n, kc: (k, kc, n)),
            pl.BlockSpec((BM, 1), lambda k, m, n, kc: (m, 0)),
        ],
        out_specs=pl.BlockSpec((1, 1, BM, 128),
                               lambda k, m, n, kc: (k, n, m, 0)),
        out_shape=jax.ShapeDtypeStruct((KS, NC, N, HC), jnp.float32),
        scratch_shapes=[pltpu.VMEM((BM, 128), jnp.float32)],
    )(agg, root, w, dis)


def _convout_kernel(a_ref, r_ref, d_ref, o_ref):
    d = d_ref[...]
    v = jnp.maximum(d * a_ref[:, 0] + r_ref[:, 0], 0.0)
    o_ref[0] = jnp.maximum(jnp.mean(v, axis=0), 0.0)


def _convout(agg, root, dis):
    return pl.pallas_call(
        _convout_kernel,
        grid=(NC, MB),
        in_specs=[
            pl.BlockSpec((KS, 1, BM, 128), lambda c, m: (0, c, m, 0)),
            pl.BlockSpec((KS, 1, BM, 128), lambda c, m: (0, c, m, 0)),
            pl.BlockSpec((BM, 1), lambda c, m: (m, 0)),
        ],
        out_specs=pl.BlockSpec((1, BM, 128), lambda c, m: (c, m, 0)),
        out_shape=jax.ShapeDtypeStruct((NC, N, HC), jnp.float32),
    )(agg, root, dis)


def _head_kernel(x_ref, w1_ref, b1_ref, batch_ref,
                 bn1g, bn1b, w2, b2, bn2g, bn2b, w3, b3, bn3g, bn3b, wo, bo,
                 o_ref, acc, gmax):
    m = pl.program_id(0)
    kc = pl.program_id(1)

    @pl.when(kc == 0)
    def _():
        acc[...] = jnp.dot(x_ref[0], w1_ref[0],
                           preferred_element_type=jnp.float32)

    @pl.when(kc == 1)
    def _():
        h = jnp.maximum(
            acc[...]
            + jnp.dot(x_ref[0], w1_ref[0], preferred_element_type=jnp.float32)
            + b1_ref[...], 0.0)

        @pl.when(m == 0)
        def _():
            gmax[...] = jnp.full_like(gmax, -3.0e38)

        b = batch_ref[...]
        glo = jnp.min(b)
        ghi = jnp.max(b)

        def gbody(g, carry):
            mask = b == g
            val = jnp.max(jnp.where(mask, h, -3.0e38), axis=0, keepdims=True)
            gmax[pl.ds(g, 1), :] = jnp.maximum(gmax[pl.ds(g, 1), :], val)
            return carry
        lax.fori_loop(glo, ghi + 1, gbody, 0)

        @pl.when(m == MB - 1)
        def _():
            eps = 1e-5
            g = gmax[...]

            def bn(v, gam, bet):
                mu = jnp.mean(v, axis=0, keepdims=True)
                var = jnp.mean((v - mu) ** 2, axis=0, keepdims=True)
                return (v - mu) / jnp.sqrt(var + eps) * gam + bet

            g = bn(g, bn1g[...], bn1b[...])
            g = jnp.maximum(
                jnp.dot(g, w2[...], preferred_element_type=jnp.float32)
                + b2[...], 0.0)
            g = bn(g, bn2g[...], bn2b[...])
            g = jnp.maximum(
                jnp.dot(g, w3[...], preferred_element_type=jnp.float32)
                + b3[...], 0.0)
            g = bn(g, bn3g[...], bn3b[...])
            logits = jnp.dot(g, wo[...],
                             preferred_element_type=jnp.float32) + bo[...]
            mx = jnp.max(logits, axis=-1, keepdims=True)
            s = logits - mx
            lse = jnp.log(jnp.sum(jnp.exp(s), axis=-1, keepdims=True))
            o_ref[...] = s - lse


def _head(x2s, l1w3, l1b, batch2d, bn1g, bn1b, l2w, l2b, bn2g, bn2b,
          l3w, l3b, bn3g, bn3b, low, lob):
    def full(shape):
        return pl.BlockSpec(shape, lambda m, kc, _s=shape: (0,) * len(_s))
    return pl.pallas_call(
        _head_kernel,
        grid=(MB, 2),
        in_specs=[
            pl.BlockSpec((1, BM, 128), lambda m, kc: (kc, m, 0)),
            pl.BlockSpec((1, 128, H), lambda m, kc: (kc, 0, 0)),
            full((1, H)),
            pl.BlockSpec((BM, 1), lambda m, kc: (m, 0)),
            full((1, H)), full((1, H)), full((H, H)), full((1, H)),
            full((1, H)), full((1, H)), full((H, H)), full((1, H)),
            full((1, H)), full((1, H)), full((H, C)), full((1, C)),
        ],
        out_specs=full((G, C)),
        out_shape=jax.ShapeDtypeStruct((G, C), jnp.float32),
        scratch_shapes=[pltpu.VMEM((BM, H), jnp.float32),
                        pltpu.VMEM((G, H), jnp.float32)],
    )(x2s, l1w3, l1b, batch2d, bn1g, bn1b, l2w, l2b, bn2g, bn2b,
      l3w, l3b, bn3g, bn3b, low, lob)


def _mk_wcat(init_w, root_w, fin):
    wi = jnp.moveaxis(init_w, 0, 1).reshape(fin, KS * H)
    wr = jnp.moveaxis(root_w.reshape(TL * KS, fin, H), 1, 0)
    wr = wr.reshape(fin, TL * KS * H)
    return jnp.concatenate([wi, wr], axis=1)


def _mk_bcat(b):
    br = b.reshape(TL * KS * 2, 128)
    return jnp.concatenate([jnp.zeros((2 * KS, 128), jnp.float32), br],
                           axis=0).reshape(24, 1, 128)


def _conv(front24, w_stack, dis, row3d, colp, ew3d):
    hsc = front24[0:2 * KS].reshape(KS, NC, N, HC)
    roots = front24[2 * KS:].reshape(TL, KS, NC, N, HC)
    for t in range(TL):
        agg = _prop_call(hsc, row3d, colp, ew3d)
        if t < TL - 1:
            hsc = _layer_mm(agg, roots[t], w_stack[t], dis)
        else:
            return _convout(agg, roots[t], dis)


def kernel(x, edge_index, edge_attr, batch, c1_init_w, c1_w, c1_root_w, c1_b,
           c2_init_w, c2_w, c2_root_w, c2_b, l1_w, l1_b, bn1_g, bn1_b,
           l2_w, l2_b, bn2_g, bn2_b, l3_w, l3_b, bn3_g, bn3_b, lo_w, lo_b):
    f32 = jnp.float32
    pad = EP - E
    row1 = jnp.pad(edge_index[0], (0, pad))
    col1 = jnp.pad(edge_index[1], (0, pad))
    ew1 = jnp.pad(edge_attr, (0, pad))
    row3d = row1.reshape(NSUPT, 8, 128)
    col3d = col1.reshape(NSUPT, 8, 128)
    ew3d = ew1.reshape(NSUPT, 8, 128)
    colp = jnp.stack([jnp.where(col1 < PH, col1, PH),
                      jnp.where(col1 >= PH, col1 - PH, PH)]
                     ).reshape(2, NSUPT, 8, 128)

    deg2 = _deg_call(col3d, ew3d)
    dis = _dis(deg2.reshape(NC, N))

    x4p = jnp.pad(x[:, :FIN], ((0, 0), (0, 4))).reshape(1, N, 8)
    w1cat = jnp.pad(_mk_wcat(c1_init_w, c1_root_w, FIN),
                    ((0, 4), (0, 0))).reshape(1, 8, 24 * 128)
    f1 = _front(x4p, w1cat, _mk_bcat(c1_b), dis)
    x1s = _conv(f1, c1_w, dis, row3d, colp, ew3d)

    w2cat = _mk_wcat(c2_init_w, c2_root_w, H).reshape(2, 128, 24 * 128)
    f2 = _front(x1s, w2cat, _mk_bcat(c2_b), dis)
    x2s = _conv(f2, c2_w, dis, row3d, colp, ew3d)

    out = _head(x2s, l1_w.reshape(2, 128, H), l1_b.reshape(1, H),
                batch.reshape(N, 1),
                bn1_g.reshape(1, H), bn1_b.reshape(1, H),
                l2_w, l2_b.reshape(1, H),
                bn2_g.reshape(1, H), bn2_b.reshape(1, H),
                l3_w, l3_b.reshape(1, H),
                bn3_g.reshape(1, H), bn3_b.reshape(1, H),
                lo_w, lo_b.reshape(1, C))
    return out
